# baseline (speedup 1.0000x reference)
import functools
import math

import jax
import jax.numpy as jnp
import numpy as np
from jax import lax
from jax.experimental import pallas as pl
from jax.experimental.pallas import tpu as pltpu
from jax.experimental.pallas import tpu_sc as plsc

_B, _S, _V = 4, 512, 32000
_N = _B * _S
_LS = 0.1
_PAD = 0
_TRG_FACTOR = 1.0 - _LS

_U = float(np.float32(_LS / (_V - 2)))
_C1 = float(np.float32(_U) * np.float32(np.log(np.float32(_U))))

_BR = 256
_BV = 6400

_NC, _NS = 2, 16
_NW = _NC * _NS
_RPW = _N // _NW
_L = 16


def _xlogx(d):
    safe = jnp.where(d > 0, d, 1.0)
    return d * jnp.log(safe)


def _stream_body(trg_ref, pred_ref, out_ref):
    t = trg_ref[...]
    p = pred_ref[...]
    out_ref[...] = jnp.where(t == _PAD, 0.0, _C1 - _U * p)


def _stream(trg_col, pred2):
    grid = (_N // _BR, _V // _BV)
    return pl.pallas_call(
        _stream_body,
        grid=grid,
        in_specs=[
            pl.BlockSpec((_BR, 1), lambda i, j: (i, 0)),
            pl.BlockSpec((_BR, _BV), lambda i, j: (i, j)),
        ],
        out_specs=pl.BlockSpec((_BR, _BV), lambda i, j: (i, j)),
        out_shape=jax.ShapeDtypeStruct((_N, _V), jnp.float32),
        compiler_params=pltpu.CompilerParams(
            dimension_semantics=("parallel", "parallel"),
        ),
    )(trg_col, pred2)


def _fix_body(trg_ref, bt_ref, off_ref, f_ref):
    t = trg_ref[...]
    bt = bt_ref[...]
    off = off_ref[...]

    a = _TRG_FACTOR * (1.0 - off)
    o = off * _TRG_FACTOR
    live = t != _PAD

    d_b = jnp.where(live, _U + o, 0.0)
    d_t = jnp.where(live, a + jnp.where(bt == t, o, 0.0), 0.0)
    d_0 = jnp.where(live & (bt == _PAD), o, 0.0)
    z = jnp.zeros_like(off)
    f_ref[...] = jnp.concatenate(
        [d_b, _xlogx(d_b), d_t, _xlogx(d_t), d_0, _xlogx(d_0), z, z], axis=0
    )


def _fix_table(trg_row, bt_row, off_row):
    return pl.pallas_call(
        _fix_body,
        in_specs=[pl.BlockSpec((1, _N), lambda: (0, 0))] * 3,
        out_specs=pl.BlockSpec((8, _N), lambda: (0, 0)),
        out_shape=jax.ShapeDtypeStruct((8, _N), jnp.float32),
    )(trg_row, bt_row, off_row)


def _sc_body(trg_hbm, bt_hbm, f_hbm, pred_hbm, out_hbm,
             t_v, bt_v, db_v, gb_v, dt_v, gt_v, d0_v, g0_v,
             ib_v, it_v, i0_v, pb_v, pt_v, p0_v, vb_v, vt_v, v0_v, sem):
    wid = lax.axis_index("s") * _NC + lax.axis_index("c")
    base = wid * _RPW

    pltpu.sync_copy(trg_hbm.at[pl.ds(base, _RPW)], t_v)
    pltpu.sync_copy(bt_hbm.at[pl.ds(base, _RPW)], bt_v)
    pltpu.sync_copy(f_hbm.at[0, pl.ds(base, _RPW)], db_v)
    pltpu.sync_copy(f_hbm.at[1, pl.ds(base, _RPW)], gb_v)
    pltpu.sync_copy(f_hbm.at[2, pl.ds(base, _RPW)], dt_v)
    pltpu.sync_copy(f_hbm.at[3, pl.ds(base, _RPW)], gt_v)
    pltpu.sync_copy(f_hbm.at[4, pl.ds(base, _RPW)], d0_v)
    pltpu.sync_copy(f_hbm.at[5, pl.ds(base, _RPW)], g0_v)

    for k in range(_RPW // _L):
        sl = pl.ds(k * _L, _L)
        rowbase = (base + k * _L + lax.iota(jnp.int32, _L)) * _V
        i0_v[sl] = rowbase
        it_v[sl] = rowbase + t_v[sl]
        ib_v[sl] = rowbase + bt_v[sl]

    pltpu.async_copy(pred_hbm.at[ib_v], pb_v, sem).wait()
    pltpu.async_copy(pred_hbm.at[it_v], pt_v, sem).wait()
    pltpu.async_copy(pred_hbm.at[i0_v], p0_v, sem).wait()

    for k in range(_RPW // _L):
        sl = pl.ds(k * _L, _L)
        vb_v[sl] = gb_v[sl] - db_v[sl] * pb_v[sl]
        vt_v[sl] = gt_v[sl] - dt_v[sl] * pt_v[sl]
        v0_v[sl] = g0_v[sl] - d0_v[sl] * p0_v[sl]

    pltpu.async_copy(vb_v, out_hbm.at[ib_v], sem).wait()
    pltpu.async_copy(vt_v, out_hbm.at[it_v], sem).wait()
    pltpu.async_copy(v0_v, out_hbm.at[i0_v], sem).wait()


def _sc_fix(out_ref, trg_flat, bt_flat, ftab, pred_flat):
    mesh = plsc.VectorSubcoreMesh(core_axis_name="c", subcore_axis_name="s")
    f = functools.partial(
        pl.kernel,
        mesh=mesh,
        out_type=(),
        scratch_types=[
            pltpu.VMEM((_RPW,), jnp.int32),
            pltpu.VMEM((_RPW,), jnp.int32),
        ] + [pltpu.VMEM((_RPW,), jnp.float32)] * 6
          + [pltpu.VMEM((_RPW,), jnp.int32)] * 3
          + [pltpu.VMEM((_RPW,), jnp.float32)] * 6
          + [pltpu.SemaphoreType.DMA],
    )(_sc_body)
    f(trg_flat, bt_flat, ftab, pred_flat, out_ref)


@jax.jit
def kernel(pred, trg, biased_trg, biased_offset):
    pred2 = pred.reshape(_N, _V)
    trg_col = trg.reshape(_N, 1)
    trg_row = trg.reshape(1, _N)
    bt_row = biased_trg.reshape(1, _N)
    off_row = biased_offset.reshape(1, _N)

    ftab = _fix_table(trg_row, bt_row, off_row)
    out = _stream(trg_col, pred2)
    out_ref = jax.new_ref(out.reshape(_N * _V))
    _sc_fix(
        out_ref,
        trg.reshape(_N),
        biased_trg.reshape(_N),
        ftab,
        pred.reshape(_N * _V),
    )
    return out_ref[...].reshape(_N, _V)


# speedup vs baseline: 1.9716x; 1.9716x over previous
import functools
import math

import jax
import jax.numpy as jnp
import numpy as np
from jax import lax
from jax.experimental import pallas as pl
from jax.experimental.pallas import tpu as pltpu
from jax.experimental.pallas import tpu_sc as plsc

_B, _S, _V = 4, 512, 32000
_N = _B * _S
_LS = 0.1
_PAD = 0
_TRG_FACTOR = 1.0 - _LS

_U = float(np.float32(_LS / (_V - 2)))
_C1 = float(np.float32(_U) * np.float32(np.log(np.float32(_U))))

_BR = 256
_BV = 6400

_NC, _NS = 2, 16
_NW = _NC * _NS
_RPW = _N // _NW
_L = 16


def _xlogx(d):
    safe = jnp.where(d > 0, d, 1.0)
    return d * jnp.log(safe)


def _fix_body(trg_ref, bt_ref, off_ref, f_ref):
    t = trg_ref[...]
    bt = bt_ref[...]
    off = off_ref[...]

    a = _TRG_FACTOR * (1.0 - off)
    o = off * _TRG_FACTOR
    live = t != _PAD

    d_b = jnp.where(live, _U + o, 0.0)
    d_t = jnp.where(live, a + jnp.where(bt == t, o, 0.0), 0.0)
    d_0 = jnp.where(live & (bt == _PAD), o, 0.0)
    z = jnp.zeros_like(off)
    f_ref[...] = jnp.concatenate(
        [d_b, _xlogx(d_b), d_t, _xlogx(d_t), d_0, _xlogx(d_0), z, z], axis=0
    )


def _fix_table(trg_row, bt_row, off_row):
    return pl.pallas_call(
        _fix_body,
        in_specs=[pl.BlockSpec((1, _N), lambda: (0, 0))] * 3,
        out_specs=pl.BlockSpec((8, _N), lambda: (0, 0)),
        out_shape=jax.ShapeDtypeStruct((8, _N), jnp.float32),
    )(trg_row, bt_row, off_row)


def _sc_body(trg_hbm, bt_hbm, f_hbm, pred_hbm, v_hbm,
             t_v, bt_v, db_v, gb_v, dt_v, gt_v, d0_v, g0_v,
             ib_v, it_v, i0_v, pb_v, pt_v, p0_v, vb_v, vt_v, v0_v, sem):
    wid = lax.axis_index("s") * _NC + lax.axis_index("c")
    base = wid * _RPW

    pltpu.sync_copy(trg_hbm.at[pl.ds(base, _RPW)], t_v)
    pltpu.sync_copy(bt_hbm.at[pl.ds(base, _RPW)], bt_v)
    pltpu.sync_copy(f_hbm.at[0, pl.ds(base, _RPW)], db_v)
    pltpu.sync_copy(f_hbm.at[1, pl.ds(base, _RPW)], gb_v)
    pltpu.sync_copy(f_hbm.at[2, pl.ds(base, _RPW)], dt_v)
    pltpu.sync_copy(f_hbm.at[3, pl.ds(base, _RPW)], gt_v)
    pltpu.sync_copy(f_hbm.at[4, pl.ds(base, _RPW)], d0_v)
    pltpu.sync_copy(f_hbm.at[5, pl.ds(base, _RPW)], g0_v)

    for k in range(_RPW // _L):
        sl = pl.ds(k * _L, _L)
        rowbase = (base + k * _L + lax.iota(jnp.int32, _L)) * _V
        i0_v[sl] = rowbase
        it_v[sl] = rowbase + t_v[sl]
        ib_v[sl] = rowbase + bt_v[sl]

    pltpu.async_copy(pred_hbm.at[ib_v], pb_v, sem).wait()
    pltpu.async_copy(pred_hbm.at[it_v], pt_v, sem).wait()
    pltpu.async_copy(pred_hbm.at[i0_v], p0_v, sem).wait()

    for k in range(_RPW // _L):
        sl = pl.ds(k * _L, _L)
        vb_v[sl] = gb_v[sl] - db_v[sl] * pb_v[sl]
        vt_v[sl] = gt_v[sl] - dt_v[sl] * pt_v[sl]
        v0_v[sl] = g0_v[sl] - d0_v[sl] * p0_v[sl]

    pltpu.sync_copy(vb_v, v_hbm.at[0, pl.ds(base, _RPW)])
    pltpu.sync_copy(vt_v, v_hbm.at[1, pl.ds(base, _RPW)])
    pltpu.sync_copy(v0_v, v_hbm.at[2, pl.ds(base, _RPW)])


def _sc_values(trg_flat, bt_flat, ftab, pred_flat):
    mesh = plsc.VectorSubcoreMesh(core_axis_name="c", subcore_axis_name="s")
    f = functools.partial(
        pl.kernel,
        mesh=mesh,
        out_type=jax.ShapeDtypeStruct((3, _N), jnp.float32),
        scratch_types=[
            pltpu.VMEM((_RPW,), jnp.int32),
            pltpu.VMEM((_RPW,), jnp.int32),
        ] + [pltpu.VMEM((_RPW,), jnp.float32)] * 6
          + [pltpu.VMEM((_RPW,), jnp.int32)] * 3
          + [pltpu.VMEM((_RPW,), jnp.float32)] * 6
          + [pltpu.SemaphoreType.DMA],
    )(_sc_body)
    return f(trg_flat, bt_flat, ftab, pred_flat)


def _stream_body(trg_ref, bt_ref, vb_ref, vt_ref, v0_ref, pred_ref, out_ref):
    j = pl.program_id(1)
    t = trg_ref[...]
    bt = bt_ref[...]
    vb = vb_ref[...]
    vt = vt_ref[...]
    v0 = v0_ref[...]

    live = t != _PAD
    c1r = jnp.where(live, _C1, 0.0)
    ur = jnp.where(live, _U, 0.0)

    p = pred_ref[...]
    col = jax.lax.broadcasted_iota(jnp.int32, p.shape, 1) + j * _BV

    out = c1r - ur * p
    out = jnp.where(col == bt, vb, out)
    out = jnp.where(col == t, vt, out)
    out = jnp.where(col == _PAD, v0, out)
    out_ref[...] = out


def _stream(trg_col, bt_col, vb, vt, v0, pred2):
    grid = (_N // _BR, _V // _BV)
    row_spec = pl.BlockSpec((_BR, 1), lambda i, j: (i, 0))
    return pl.pallas_call(
        _stream_body,
        grid=grid,
        in_specs=[row_spec] * 5 + [pl.BlockSpec((_BR, _BV), lambda i, j: (i, j))],
        out_specs=pl.BlockSpec((_BR, _BV), lambda i, j: (i, j)),
        out_shape=jax.ShapeDtypeStruct((_N, _V), jnp.float32),
        compiler_params=pltpu.CompilerParams(
            dimension_semantics=("parallel", "parallel"),
        ),
    )(trg_col, bt_col, vb, vt, v0, pred2)


@jax.jit
def kernel(pred, trg, biased_trg, biased_offset):
    pred2 = pred.reshape(_N, _V)
    trg_row = trg.reshape(1, _N)
    bt_row = biased_trg.reshape(1, _N)
    off_row = biased_offset.reshape(1, _N)

    ftab = _fix_table(trg_row, bt_row, off_row)
    vals = _sc_values(
        trg.reshape(_N), biased_trg.reshape(_N), ftab, pred.reshape(_N * _V)
    )
    vb = vals[0].reshape(_N, 1)
    vt = vals[1].reshape(_N, 1)
    v0 = vals[2].reshape(_N, 1)
    return _stream(
        trg.reshape(_N, 1), biased_trg.reshape(_N, 1), vb, vt, v0, pred2
    )
